# baseline (speedup 1.0000x reference)
import jax
import jax.numpy as jnp
from jax.experimental import pallas as pl
from jax.experimental.pallas import tpu as pltpu

_B, _SEQ, _H1, _H2, _H3 = 16, 512, 128, 256, 256
_G = 8
_PREC = jax.lax.Precision.DEFAULT


def _gcn3_kernel(x_ref, a_ref, w1_ref, b1_ref, w2_ref, b2_ref, w3_ref,
                 b3_ref, o_ref, h3_ref):
    i = pl.program_id(0)

    r = jax.lax.broadcasted_iota(jnp.int32, (_SEQ, _SEQ), 0)
    c = jax.lax.broadcasted_iota(jnp.int32, (_SEQ, _SEQ), 1)
    eye = r == c

    def per_graph(g, carry):
        A = a_ref[g]
        diag = jnp.sum(jnp.where(eye, A, 0.0), axis=0, keepdims=True)
        new_diag = jnp.where(diag != 0.0, diag, 1.0)
        A_hat = jnp.where(eye, jnp.broadcast_to(new_diag, (_SEQ, _SEQ)), A)
        deg = jnp.sum(A_hat, axis=0, keepdims=True)
        dinv = jnp.where(deg > 0.0, jax.lax.rsqrt(deg), 0.0)
        dinv_col = jnp.sum(
            jnp.where(eye, jnp.broadcast_to(dinv, (_SEQ, _SEQ)), 0.0),
            axis=1, keepdims=True)
        An = dinv_col * A_hat * dinv

        def prop(u):
            return jax.lax.dot_general(
                An, u, (((0,), (0,)), ((), ())),
                preferred_element_type=jnp.float32, precision=_PREC)

        bsel = jax.lax.broadcasted_iota(jnp.int32, (_B, 1), 0) == (i * _G + g)
        x = jnp.sum(jnp.where(bsel, x_ref[...], 0.0), axis=0, keepdims=True)
        v1 = jax.lax.dot_general(
            x, An, (((1,), (0,)), ((), ())),
            preferred_element_type=jnp.float32, precision=_PREC)
        h1 = jax.lax.dot_general(
            v1, w1_ref[...], (((0,), (0,)), ((), ())),
            preferred_element_type=jnp.float32, precision=_PREC) + b1_ref[...]
        xw2 = jnp.dot(h1, w2_ref[...], preferred_element_type=jnp.float32,
                      precision=_PREC)
        h2 = jnp.maximum(prop(xw2) + b2_ref[...], 0.0)
        xw3 = jnp.dot(h2, w3_ref[...], preferred_element_type=jnp.float32,
                      precision=_PREC)
        h3_ref[g] = prop(xw3) + b3_ref[...]
        return carry

    jax.lax.fori_loop(0, _G, per_graph, 0, unroll=True)

    def per_row(s, carry):
        o_ref[:, pl.ds(s * _H3, _H3)] = h3_ref[:, s, :]
        return carry

    jax.lax.fori_loop(0, _SEQ, per_row, 0)


def kernel(x, adj, W1, b1, W2, b2, W3, b3):
    return pl.pallas_call(
        _gcn3_kernel,
        grid=(_B // _G,),
        in_specs=[
            pl.BlockSpec((_B, _SEQ), lambda i: (0, 0)),
            pl.BlockSpec((_G, _SEQ, _SEQ), lambda i: (i, 0, 0)),
            pl.BlockSpec((1, _H1), lambda i: (0, 0)),
            pl.BlockSpec((_H1,), lambda i: (0,)),
            pl.BlockSpec((_H1, _H2), lambda i: (0, 0)),
            pl.BlockSpec((_H2,), lambda i: (0,)),
            pl.BlockSpec((_H2, _H3), lambda i: (0, 0)),
            pl.BlockSpec((_H3,), lambda i: (0,)),
        ],
        out_specs=pl.BlockSpec((_G, _SEQ * _H3), lambda i: (i, 0)),
        out_shape=jax.ShapeDtypeStruct((_B, _SEQ * _H3), jnp.float32),
        scratch_shapes=[pltpu.VMEM((_G, _SEQ, _H3), jnp.float32)],
        compiler_params=pltpu.CompilerParams(
            dimension_semantics=("arbitrary",)),
    )(x, adj, W1, b1, W2, b2, W3, b3)


# speedup vs baseline: 1.5239x; 1.5239x over previous
import jax
import jax.numpy as jnp
from jax.experimental import pallas as pl
from jax.experimental.pallas import tpu as pltpu

_B, _SEQ, _H1, _H2, _H3 = 16, 512, 128, 256, 256
_G = 8
_PREC = jax.lax.Precision.DEFAULT


def _gcn3_kernel(x_ref, a_ref, w1_ref, b1_ref, w2_ref, b2_ref, w3_ref,
                 b3_ref, o_ref, h3_ref):
    i = pl.program_id(0)

    r = jax.lax.broadcasted_iota(jnp.int32, (_SEQ, _SEQ), 0)
    c = jax.lax.broadcasted_iota(jnp.int32, (_SEQ, _SEQ), 1)
    eye = r == c

    def per_graph(g, carry):
        A = a_ref[g]
        diag = jnp.sum(jnp.where(eye, A, 0.0), axis=0, keepdims=True)
        new_diag = jnp.where(diag != 0.0, diag, 1.0)
        A_hat = jnp.where(eye, jnp.broadcast_to(new_diag, (_SEQ, _SEQ)), A)
        deg = jnp.sum(A_hat, axis=0, keepdims=True)
        dinv = jnp.where(deg > 0.0, jax.lax.rsqrt(deg), 0.0)
        dinv_col = jnp.sum(
            jnp.where(eye, jnp.broadcast_to(dinv, (_SEQ, _SEQ)), 0.0),
            axis=1, keepdims=True)
        An = dinv_col * A_hat * dinv

        def prop(u):
            return jax.lax.dot_general(
                An, u, (((0,), (0,)), ((), ())),
                preferred_element_type=jnp.float32, precision=_PREC)

        bsel = jax.lax.broadcasted_iota(jnp.int32, (_B, 1), 0) == (i * _G + g)
        x = jnp.sum(jnp.where(bsel, x_ref[...], 0.0), axis=0, keepdims=True)
        v1 = jax.lax.dot_general(
            x, An, (((1,), (0,)), ((), ())),
            preferred_element_type=jnp.float32, precision=_PREC)
        h1 = jax.lax.dot_general(
            v1, w1_ref[...], (((0,), (0,)), ((), ())),
            preferred_element_type=jnp.float32, precision=_PREC) + b1_ref[...]
        xw2 = jnp.dot(h1, w2_ref[...], preferred_element_type=jnp.float32,
                      precision=_PREC)
        h2 = jnp.maximum(prop(xw2) + b2_ref[...], 0.0)
        xw3 = jnp.dot(h2, w3_ref[...], preferred_element_type=jnp.float32,
                      precision=_PREC)
        h3_ref[g] = prop(xw3) + b3_ref[...]
        return carry

    jax.lax.fori_loop(0, _G, per_graph, 0, unroll=True)

    o_ref[...] = h3_ref[...].reshape(_G, _SEQ * _H3)


def kernel(x, adj, W1, b1, W2, b2, W3, b3):
    return pl.pallas_call(
        _gcn3_kernel,
        grid=(_B // _G,),
        in_specs=[
            pl.BlockSpec((_B, _SEQ), lambda i: (0, 0)),
            pl.BlockSpec((_G, _SEQ, _SEQ), lambda i: (i, 0, 0)),
            pl.BlockSpec((1, _H1), lambda i: (0, 0)),
            pl.BlockSpec((_H1,), lambda i: (0,)),
            pl.BlockSpec((_H1, _H2), lambda i: (0, 0)),
            pl.BlockSpec((_H2,), lambda i: (0,)),
            pl.BlockSpec((_H2, _H3), lambda i: (0, 0)),
            pl.BlockSpec((_H3,), lambda i: (0,)),
        ],
        out_specs=pl.BlockSpec((_G, _SEQ * _H3), lambda i: (i, 0)),
        out_shape=jax.ShapeDtypeStruct((_B, _SEQ * _H3), jnp.float32),
        scratch_shapes=[pltpu.VMEM((_G, _SEQ, _H3), jnp.float32)],
        compiler_params=pltpu.CompilerParams(
            dimension_semantics=("arbitrary",)),
    )(x, adj, W1, b1, W2, b2, W3, b3)


# speedup vs baseline: 1.7333x; 1.1374x over previous
import jax
import jax.numpy as jnp
from jax.experimental import pallas as pl
from jax.experimental.pallas import tpu as pltpu

_B, _SEQ, _H1, _H2, _H3 = 16, 512, 128, 256, 256
_G = 8
_PREC = jax.lax.Precision.DEFAULT


def _dot(a, b, dims, prec=_PREC):
    return jax.lax.dot_general(a, b, (dims, ((), ())),
                               preferred_element_type=jnp.float32,
                               precision=prec)


def _gcn3_kernel(x_ref, a_ref, w1_ref, b1_ref, w2_ref, b2_ref, w3_ref,
                 b3_ref, o_ref, h3_ref):
    i = pl.program_id(0)

    r = jax.lax.broadcasted_iota(jnp.int32, (_SEQ, _SEQ), 0)
    c = jax.lax.broadcasted_iota(jnp.int32, (_SEQ, _SEQ), 1)
    eye = r == c
    ones_row = jnp.ones((1, _SEQ), jnp.float32)

    w12 = _dot(w1_ref[...], w2_ref[...], (((1,), (0,))),
               prec=jax.lax.Precision.HIGHEST)
    b1w2 = _dot(b1_ref[...].reshape(1, _H1), w2_ref[...], (((1,), (0,))),
                prec=jax.lax.Precision.HIGHEST)
    fac2 = jnp.concatenate([w12, b1w2], axis=0)

    def per_graph(g, carry):
        A = a_ref[g]
        diag = jnp.sum(jnp.where(eye, A, 0.0), axis=0, keepdims=True)
        new_diag = jnp.where(diag != 0.0, diag, 1.0)
        A_hat = jnp.where(eye, jnp.broadcast_to(new_diag, (_SEQ, _SEQ)), A)
        deg = jnp.sum(A_hat, axis=0, keepdims=True)
        dinv = jnp.where(deg > 0.0, jax.lax.rsqrt(deg), 0.0)
        dinv_col = jnp.sum(
            jnp.where(eye, jnp.broadcast_to(dinv, (_SEQ, _SEQ)), 0.0),
            axis=1, keepdims=True)

        bsel = jax.lax.broadcasted_iota(jnp.int32, (_B, 1), 0) == (i * _G + g)
        x = jnp.sum(jnp.where(bsel, x_ref[...], 0.0), axis=0, keepdims=True)

        xs = jnp.concatenate([x * dinv, dinv], axis=0)
        v1u = _dot(xs, A_hat, (((1,), (0,)))) * dinv
        v1 = v1u[0:1]
        u = v1u[1:2]
        w = _dot(v1 * dinv, A_hat, (((1,), (0,)))) * dinv

        wu = jnp.concatenate([w, u], axis=0)
        h2 = jnp.maximum(_dot(wu, fac2, (((0,), (0,)))) + b2_ref[...], 0.0)

        xw3 = _dot(h2, w3_ref[...], (((1,), (0,)))) * dinv_col
        h3_ref[g] = (_dot(A_hat, xw3, (((0,), (0,)))) * dinv_col
                     + b3_ref[...])
        return carry

    jax.lax.fori_loop(0, _G, per_graph, 0, unroll=True)

    o_ref[...] = h3_ref[...].reshape(_G, _SEQ * _H3)


def kernel(x, adj, W1, b1, W2, b2, W3, b3):
    return pl.pallas_call(
        _gcn3_kernel,
        grid=(_B // _G,),
        in_specs=[
            pl.BlockSpec((_B, _SEQ), lambda i: (0, 0)),
            pl.BlockSpec((_G, _SEQ, _SEQ), lambda i: (i, 0, 0)),
            pl.BlockSpec((1, _H1), lambda i: (0, 0)),
            pl.BlockSpec((_H1,), lambda i: (0,)),
            pl.BlockSpec((_H1, _H2), lambda i: (0, 0)),
            pl.BlockSpec((_H2,), lambda i: (0,)),
            pl.BlockSpec((_H2, _H3), lambda i: (0, 0)),
            pl.BlockSpec((_H3,), lambda i: (0,)),
        ],
        out_specs=pl.BlockSpec((_G, _SEQ * _H3), lambda i: (i, 0)),
        out_shape=jax.ShapeDtypeStruct((_B, _SEQ * _H3), jnp.float32),
        scratch_shapes=[pltpu.VMEM((_G, _SEQ, _H3), jnp.float32)],
        compiler_params=pltpu.CompilerParams(
            dimension_semantics=("arbitrary",)),
    )(x, adj, W1, b1, W2, b2, W3, b3)
